# Initial kernel scaffold; baseline (speedup 1.0000x reference)
import jax
import jax.numpy as jnp
from jax.experimental import pallas as pl


def kernel(inputs, emb_table, W, b):
    raise NotImplementedError("write your pallas kernel here")


# speedup vs baseline: 1.6797x; 1.6797x over previous
import functools

import jax
import jax.numpy as jnp
from jax import lax
from jax.experimental import pallas as pl
from jax.experimental.pallas import tpu as pltpu
from jax.experimental.pallas import tpu_sc as plsc

VOCAB = 1_000_000
EMBED = 32
BATCH = 16384
HIST = 50

NC = 2
NS = 16
NW = NC * NS
BPW = BATCH // NW
CHUNK = 128
CPH = BPW // CHUNK
NCHUNK = HIST * CPH

def _sc_body(table_hbm, idx_hbm, out_hbm, idx_v, rows_v, acc_v, sem):
    wid = lax.axis_index("s") * NC + lax.axis_index("c")
    pltpu.sync_copy(idx_hbm.at[wid], idx_v)

    def h_body(h, carry):
        a0 = jnp.zeros((16,), jnp.float32)
        a1 = jnp.zeros((16,), jnp.float32)
        for u in range(CPH):
            c = h * CPH + u
            pltpu.async_copy(table_hbm.at[idx_v.at[c]], rows_v, sem).wait()

            def r_body(r, accs, _rows=rows_v):
                b0, b1 = accs
                return (b0 + _rows[r, pl.ds(0, 16)],
                        b1 + _rows[r, pl.ds(16, 16)])

            a0, a1 = lax.fori_loop(0, CHUNK, r_body, (a0, a1))
        acc_v[h, pl.ds(0, 16)] = a0
        acc_v[h, pl.ds(16, 16)] = a1
        return carry

    lax.fori_loop(0, HIST, h_body, 0)
    pltpu.sync_copy(acc_v, out_hbm.at[wid])


@functools.cache
def _sc_gather_sum():
    mesh = plsc.VectorSubcoreMesh(
        core_axis_name="c", subcore_axis_name="s", num_cores=NC, num_subcores=NS
    )
    return pl.kernel(
        _sc_body,
        out_type=jax.ShapeDtypeStruct((NW, HIST, EMBED), jnp.float32),
        mesh=mesh,
        scratch_types=[
            pltpu.VMEM((NCHUNK, CHUNK), jnp.int32),
            pltpu.VMEM((CHUNK, EMBED), jnp.float32),
            pltpu.VMEM((HIST, EMBED), jnp.float32),
            pltpu.SemaphoreType.DMA,
        ],
        compiler_params=pltpu.CompilerParams(use_tc_tiling_on_sc=False),
    )


_VB = 8192


def _tc_body(p_ref, w_ref, b_ref, o_ref):
    s = jnp.sum(p_ref[...], axis=0)
    o_ref[...] = lax.dot_general(
        s, w_ref[...], (((1,), (1,)), ((), ())),
        preferred_element_type=jnp.float32,
    ) + b_ref[...]


def _tc_project(partials, W, b2d):
    nblk = pl.cdiv(VOCAB, _VB)
    return pl.pallas_call(
        _tc_body,
        grid=(nblk,),
        in_specs=[
            pl.BlockSpec((NW, HIST, EMBED), lambda i: (0, 0, 0)),
            pl.BlockSpec((_VB, EMBED), lambda i: (i, 0)),
            pl.BlockSpec((1, _VB), lambda i: (0, i)),
        ],
        out_specs=pl.BlockSpec((HIST, _VB), lambda i: (0, i)),
        out_shape=jax.ShapeDtypeStruct((HIST, VOCAB), jnp.float32),
    )(partials, W, b2d)


def kernel(inputs, emb_table, W, b):
    idx = inputs.astype(jnp.int32)
    idxr = (idx.reshape(NW, BPW, HIST)
               .transpose(0, 2, 1)
               .reshape(NW, NCHUNK, CHUNK))
    partials = _sc_gather_sum()(emb_table, idxr)
    return _tc_project(partials, W, b.reshape(1, VOCAB))


# speedup vs baseline: 2.0604x; 1.2267x over previous
import functools

import jax
import jax.numpy as jnp
from jax import lax
from jax.experimental import pallas as pl
from jax.experimental.pallas import tpu as pltpu
from jax.experimental.pallas import tpu_sc as plsc

VOCAB = 1_000_000
EMBED = 32
BATCH = 16384
HIST = 50

NC = 2
NS = 16
NW = NC * NS
BPW = BATCH // NW
CHUNK = 128
CPH = BPW // CHUNK
NCHUNK = HIST * CPH


def _sc_body(table_hbm, idx_hbm, out_hbm, idx_v, rows_v, acc_v, sem):
    wid = lax.axis_index("s") * NC + lax.axis_index("c")
    pltpu.sync_copy(idx_hbm.at[wid], idx_v)

    def h_body(h, carry):
        a0 = jnp.zeros((16,), jnp.float32)
        a1 = jnp.zeros((16,), jnp.float32)
        for u in range(CPH):
            c = h * CPH + u
            pltpu.async_copy(table_hbm.at[idx_v.at[c]], rows_v, sem).wait()

            def r_body(r, accs, _rows=rows_v):
                b0, b1 = accs
                return (b0 + _rows[r, pl.ds(0, 16)],
                        b1 + _rows[r, pl.ds(16, 16)])

            a0, a1 = lax.fori_loop(0, CHUNK, r_body, (a0, a1))
        acc_v[h, pl.ds(0, 16)] = a0
        acc_v[h, pl.ds(16, 16)] = a1
        return carry

    lax.fori_loop(0, HIST, h_body, 0)
    pltpu.sync_copy(acc_v, out_hbm.at[wid])


@functools.cache
def _sc_gather_sum():
    mesh = plsc.VectorSubcoreMesh(
        core_axis_name="c", subcore_axis_name="s", num_cores=NC, num_subcores=NS
    )
    return pl.kernel(
        _sc_body,
        out_type=jax.ShapeDtypeStruct((NW, HIST, EMBED), jnp.float32),
        mesh=mesh,
        scratch_types=[
            pltpu.VMEM((NCHUNK, CHUNK), jnp.int32),
            pltpu.VMEM((CHUNK, EMBED), jnp.float32),
            pltpu.VMEM((HIST, EMBED), jnp.float32),
            pltpu.SemaphoreType.DMA,
        ],
        compiler_params=pltpu.CompilerParams(use_tc_tiling_on_sc=False),
    )


_VB = 32768


def _tc_body(p_ref, wt_ref, b_ref, o_ref):
    s = jnp.sum(p_ref[...], axis=0)
    o_ref[...] = jnp.dot(
        s, wt_ref[...], preferred_element_type=jnp.float32
    ) + b_ref[...]


def _tc_project(partials, Wt, b2d):
    nblk = pl.cdiv(VOCAB, _VB)
    return pl.pallas_call(
        _tc_body,
        grid=(nblk,),
        in_specs=[
            pl.BlockSpec((NW, HIST, EMBED), lambda i: (0, 0, 0)),
            pl.BlockSpec((EMBED, _VB), lambda i: (0, i)),
            pl.BlockSpec((1, _VB), lambda i: (0, i)),
        ],
        out_specs=pl.BlockSpec((HIST, _VB), lambda i: (0, i)),
        out_shape=jax.ShapeDtypeStruct((HIST, VOCAB), jnp.float32),
    )(partials, Wt, b2d)


def kernel(inputs, emb_table, W, b):
    idx = inputs.astype(jnp.int32)
    idxr = (idx.reshape(NW, BPW, HIST)
               .transpose(0, 2, 1)
               .reshape(NW, NCHUNK, CHUNK))
    partials = _sc_gather_sum()(emb_table, idxr)
    return _tc_project(partials, W.T, b.reshape(1, VOCAB))


# speedup vs baseline: 2.6312x; 1.2770x over previous
import functools

import jax
import jax.numpy as jnp
from jax import lax
from jax.experimental import pallas as pl
from jax.experimental.pallas import tpu as pltpu
from jax.experimental.pallas import tpu_sc as plsc

VOCAB = 1_000_000
EMBED = 32
BATCH = 16384
HIST = 50

NC = 2
NS = 16
NW = NC * NS
BPW = BATCH // NW
CHUNK = 128
CPH = BPW // CHUNK
NCHUNK = HIST * CPH


NBUF = 4


def _sc_body(table_hbm, idx_hbm, out_hbm, idx_v, rows_v, acc_v, *sems):
    wid = lax.axis_index("s") * NC + lax.axis_index("c")
    pltpu.sync_copy(idx_hbm.at[wid], idx_v)

    for k in range(NBUF):
        pltpu.async_copy(table_hbm.at[idx_v.at[k]], rows_v.at[k], sems[k])

    def h_body(h, carry):
        a0 = jnp.zeros((16,), jnp.float32)
        a1 = jnp.zeros((16,), jnp.float32)
        for k in range(NBUF):
            c = h * CPH + k
            buf = rows_v.at[k]
            pltpu.make_async_copy(table_hbm.at[idx_v.at[c]], buf, sems[k]).wait()

            def r_body(r, accs, _buf=buf):
                b0, b1 = accs
                return (b0 + _buf[r, pl.ds(0, 16)],
                        b1 + _buf[r, pl.ds(16, 16)])

            a0, a1 = lax.fori_loop(0, CHUNK, r_body, (a0, a1), unroll=8)
            nc = c + NBUF

            @pl.when(nc < NCHUNK)
            def _issue(_buf=buf, _k=k, _nc=nc):
                pltpu.async_copy(table_hbm.at[idx_v.at[_nc]], _buf, sems[_k])

        acc_v[h, pl.ds(0, 16)] = a0
        acc_v[h, pl.ds(16, 16)] = a1
        return carry

    lax.fori_loop(0, HIST, h_body, 0)
    pltpu.sync_copy(acc_v, out_hbm.at[wid])


@functools.cache
def _sc_gather_sum():
    mesh = plsc.VectorSubcoreMesh(
        core_axis_name="c", subcore_axis_name="s", num_cores=NC, num_subcores=NS
    )
    return pl.kernel(
        _sc_body,
        out_type=jax.ShapeDtypeStruct((NW, HIST, EMBED), jnp.float32),
        mesh=mesh,
        scratch_types=[
            pltpu.VMEM((NCHUNK, CHUNK), jnp.int32),
            pltpu.VMEM((NBUF, CHUNK, EMBED), jnp.float32),
            pltpu.VMEM((HIST, EMBED), jnp.float32),
        ] + [pltpu.SemaphoreType.DMA] * NBUF,
        compiler_params=pltpu.CompilerParams(use_tc_tiling_on_sc=False),
    )


_VB = 32768


def _tc_body(p_ref, wt_ref, b_ref, o_ref):
    s = jnp.sum(p_ref[...], axis=0)
    o_ref[...] = jnp.dot(
        s, wt_ref[...], preferred_element_type=jnp.float32
    ) + b_ref[...]


def _tc_project(partials, Wt, b2d):
    nblk = pl.cdiv(VOCAB, _VB)
    return pl.pallas_call(
        _tc_body,
        grid=(nblk,),
        in_specs=[
            pl.BlockSpec((NW, HIST, EMBED), lambda i: (0, 0, 0)),
            pl.BlockSpec((EMBED, _VB), lambda i: (0, i)),
            pl.BlockSpec((1, _VB), lambda i: (0, i)),
        ],
        out_specs=pl.BlockSpec((HIST, _VB), lambda i: (0, i)),
        out_shape=jax.ShapeDtypeStruct((HIST, VOCAB), jnp.float32),
    )(partials, Wt, b2d)


def kernel(inputs, emb_table, W, b):
    idx = inputs.astype(jnp.int32)
    idxr = (idx.reshape(NW, BPW, HIST)
               .transpose(0, 2, 1)
               .reshape(NW, NCHUNK, CHUNK))
    partials = _sc_gather_sum()(emb_table, idxr)
    return _tc_project(partials, W.T, b.reshape(1, VOCAB))


# speedup vs baseline: 2.6397x; 1.0032x over previous
import functools

import jax
import jax.numpy as jnp
from jax import lax
from jax.experimental import pallas as pl
from jax.experimental.pallas import tpu as pltpu
from jax.experimental.pallas import tpu_sc as plsc

VOCAB = 1_000_000
EMBED = 32
BATCH = 16384
HIST = 50

NC = 2
NS = 16
NW = NC * NS
BPW = BATCH // NW
CHUNK = 128
CPH = BPW // CHUNK
NCHUNK = HIST * CPH


NBUF = 4


def _sc_body(table_hbm, idx_hbm, out_hbm, idx_v, rows_v, acc_v, *sems):
    wid = lax.axis_index("s") * NC + lax.axis_index("c")
    pltpu.sync_copy(idx_hbm.at[wid], idx_v)

    for k in range(NBUF):
        pltpu.async_copy(table_hbm.at[idx_v.at[k]], rows_v.at[k], sems[k])

    def h_body(h, carry):
        a0 = jnp.zeros((16,), jnp.float32)
        a1 = jnp.zeros((16,), jnp.float32)
        for k in range(NBUF):
            c = h * CPH + k
            buf = rows_v.at[k]
            pltpu.make_async_copy(table_hbm.at[idx_v.at[c]], buf, sems[k]).wait()

            def r_body(r, accs, _buf=buf):
                b0, b1 = accs
                return (b0 + _buf[r, pl.ds(0, 16)],
                        b1 + _buf[r, pl.ds(16, 16)])

            a0, a1 = lax.fori_loop(0, CHUNK, r_body, (a0, a1), unroll=8)
            nc = c + NBUF

            @pl.when(nc < NCHUNK)
            def _issue(_buf=buf, _k=k, _nc=nc):
                pltpu.async_copy(table_hbm.at[idx_v.at[_nc]], _buf, sems[_k])

        acc_v[h, pl.ds(0, 16)] = a0
        acc_v[h, pl.ds(16, 16)] = a1
        return carry

    lax.fori_loop(0, HIST, h_body, 0)
    pltpu.sync_copy(acc_v, out_hbm.at[wid])


@functools.cache
def _sc_gather_sum():
    mesh = plsc.VectorSubcoreMesh(
        core_axis_name="c", subcore_axis_name="s", num_cores=NC, num_subcores=NS
    )
    return pl.kernel(
        _sc_body,
        out_type=jax.ShapeDtypeStruct((NW, HIST, EMBED), jnp.float32),
        mesh=mesh,
        scratch_types=[
            pltpu.VMEM((NCHUNK, CHUNK), jnp.int32),
            pltpu.VMEM((NBUF, CHUNK, EMBED), jnp.float32),
            pltpu.VMEM((HIST, EMBED), jnp.float32),
        ] + [pltpu.SemaphoreType.DMA] * NBUF,
        compiler_params=pltpu.CompilerParams(use_tc_tiling_on_sc=False),
    )


_VB = 65536


def _tc_body(p_ref, wt_ref, b_ref, o_ref):
    s = jnp.sum(p_ref[...], axis=0)
    o_ref[...] = jnp.dot(
        s, wt_ref[...], preferred_element_type=jnp.float32
    ) + b_ref[...]


def _tc_project(partials, Wt, b2d):
    nblk = pl.cdiv(VOCAB, _VB)
    return pl.pallas_call(
        _tc_body,
        grid=(nblk,),
        in_specs=[
            pl.BlockSpec((NW, HIST, EMBED), lambda i: (0, 0, 0)),
            pl.BlockSpec((EMBED, _VB), lambda i: (0, i)),
            pl.BlockSpec((1, _VB), lambda i: (0, i)),
        ],
        out_specs=pl.BlockSpec((HIST, _VB), lambda i: (0, i)),
        out_shape=jax.ShapeDtypeStruct((HIST, VOCAB), jnp.float32),
    )(partials, Wt, b2d)


def kernel(inputs, emb_table, W, b):
    idx = inputs.astype(jnp.int32)
    idxr = (idx.reshape(NW, BPW, HIST)
               .transpose(0, 2, 1)
               .reshape(NW, NCHUNK, CHUNK))
    partials = _sc_gather_sum()(emb_table, idxr)
    return _tc_project(partials, W.T, b.reshape(1, VOCAB))
